# baseline (speedup 1.0000x reference)
import functools

import jax
import jax.numpy as jnp
from jax import lax
from jax.experimental import pallas as pl
from jax.experimental.pallas import tpu as pltpu
from jax.experimental.pallas import tpu_sc as plsc

_NC = 2
_NS = 16
_L = 16
_NW = _NC * _NS
_CHUNK = 16384

_N = 16777216
_F = 6291456
_TC_BLOCK = 524288


def _make_sc_partials(n_sc):
    per_w = n_sc // _NW
    n_chunks = per_w // _CHUNK
    assert per_w * _NW == n_sc and n_chunks * _CHUNK == per_w

    mesh = plsc.VectorSubcoreMesh(
        core_axis_name="c", subcore_axis_name="s",
        num_cores=_NC, num_subcores=_NS,
    )

    @functools.partial(
        pl.kernel,
        out_type=(
            jax.ShapeDtypeStruct((_NW, _L), jnp.int32),
            jax.ShapeDtypeStruct((_NW, _L), jnp.int32),
        ),
        mesh=mesh,
        scratch_types=[
            pltpu.VMEM((_CHUNK,), jnp.int32),
            pltpu.VMEM((_CHUNK,), jnp.int32),
            pltpu.VMEM((_CHUNK,), jnp.int32),
            pltpu.VMEM((_CHUNK,), jnp.int32),
            pltpu.VMEM((_L,), jnp.int32),
            pltpu.VMEM((_L,), jnp.int32),
            pltpu.SemaphoreType.DMA,
            pltpu.SemaphoreType.DMA,
            pltpu.SemaphoreType.DMA,
            pltpu.SemaphoreType.DMA,
        ],
    )
    def sc_partials(t_hbm, p_hbm, out_t, out_or,
                    tb0, tb1, pb0, pb1, acc_t_v, acc_or_v,
                    st0, st1, sp0, sp1):
        cid = lax.axis_index("c")
        sid = lax.axis_index("s")
        wid = sid * _NC + cid
        base = wid * per_w

        tbufs = (tb0, tb1)
        pbufs = (pb0, pb1)
        tsems = (st0, st1)
        psems = (sp0, sp1)

        def start(chunk, slot):
            off = base + chunk * _CHUNK
            dt = pltpu.async_copy(
                t_hbm.at[pl.ds(off, _CHUNK)], tbufs[slot], tsems[slot])
            dp = pltpu.async_copy(
                p_hbm.at[pl.ds(off, _CHUNK)], pbufs[slot], psems[slot])
            return dt, dp

        pending = [None, None]
        pending[0] = start(0, 0)

        z = jnp.zeros((_L,), jnp.int32)
        acc = (z, z, z, z)
        for c in range(n_chunks):
            slot = c & 1
            nxt = slot ^ 1
            if c + 1 < n_chunks:
                pending[nxt] = start(c + 1, nxt)
            dt, dp = pending[slot]
            dt.wait()
            dp.wait()

            tb = tbufs[slot]
            pb = pbufs[slot]

            def body(i, carry):
                at0, ao0, at1, ao1 = carry
                t0 = tb[pl.ds(i, _L)]
                p0 = pb[pl.ds(i, _L)]
                t1 = tb[pl.ds(i + _L, _L)]
                p1 = pb[pl.ds(i + _L, _L)]
                return (at0 + t0, ao0 + (t0 | p0),
                        at1 + t1, ao1 + (t1 | p1))

            acc = plsc.parallel_loop(
                0, _CHUNK, 2 * _L, unroll=8, carry=acc)(body)

        acc_t_v[...] = acc[0] + acc[2]
        acc_or_v[...] = acc[1] + acc[3]
        pltpu.sync_copy(acc_t_v, out_t.at[wid])
        pltpu.sync_copy(acc_or_v, out_or.at[wid])

    return sc_partials


def _tc_partials(t, p, e0):
    n_el = t.shape[0] - e0
    n_blocks = n_el // _TC_BLOCK
    assert n_blocks * _TC_BLOCK == n_el and e0 % _TC_BLOCK == 0

    def body(t_ref, p_ref, out_ref, acc_ref):
        i = pl.program_id(0)
        t_blk = t_ref[...]
        p_blk = p_ref[...]
        st = jnp.sum(t_blk)
        so = jnp.sum(t_blk | p_blk)

        @pl.when(i == 0)
        def _init():
            acc_ref[0] = st
            acc_ref[1] = so

        @pl.when(i > 0)
        def _acc():
            acc_ref[0] += st
            acc_ref[1] += so

        @pl.when(i == n_blocks - 1)
        def _out():
            out_ref[0] = acc_ref[0]
            out_ref[1] = acc_ref[1]

    blk = lambda i: (e0 // _TC_BLOCK + i,)
    return pl.pallas_call(
        body,
        grid=(n_blocks,),
        in_specs=[
            pl.BlockSpec((_TC_BLOCK,), blk),
            pl.BlockSpec((_TC_BLOCK,), blk),
        ],
        out_specs=pl.BlockSpec(memory_space=pltpu.SMEM),
        out_shape=jax.ShapeDtypeStruct((2,), jnp.int32),
        scratch_shapes=[pltpu.SMEM((2,), jnp.int32)],
    )(t, p)


def _finish(sc_pt, sc_po, tc_sums, n):
    def body(pt_ref, po_ref, tc_ref, out_ref):
        st = jnp.sum(pt_ref[...]) + tc_ref[0]
        so = jnp.sum(po_ref[...]) + tc_ref[1]
        tn = (n - so).astype(jnp.float32)
        tnfp = (n - st).astype(jnp.float32)
        out_ref[...] = jnp.full((1, 1), tn / tnfp, jnp.float32)

    out = pl.pallas_call(
        body,
        in_specs=[
            pl.BlockSpec(memory_space=pltpu.VMEM),
            pl.BlockSpec(memory_space=pltpu.VMEM),
            pl.BlockSpec(memory_space=pltpu.SMEM),
        ],
        out_shape=jax.ShapeDtypeStruct((1, 1), jnp.float32),
    )(sc_pt, sc_po, tc_sums)
    return out[0, 0]


@jax.jit
def kernel(y_true, y_pred):
    n = y_true.shape[0]
    t = y_true.astype(jnp.int32)
    p = y_pred.astype(jnp.int32)
    sc_pt, sc_po = _make_sc_partials(_F)(t, p)
    tc_sums = _tc_partials(t, p, _F)
    return _finish(sc_pt, sc_po, tc_sums, n)


# speedup vs baseline: 1.1730x; 1.1730x over previous
import functools

import jax
import jax.numpy as jnp
from jax import lax
from jax.experimental import pallas as pl
from jax.experimental.pallas import tpu as pltpu
from jax.experimental.pallas import tpu_sc as plsc

_NC = 2
_NS = 16
_L = 16
_NW = _NC * _NS
_CHUNK = 16384

_N = 16777216
_F = 10485760
_TC_BLOCK = 524288


def _make_sc_partials(n_sc):
    per_w = n_sc // _NW
    n_chunks = per_w // _CHUNK
    assert per_w * _NW == n_sc and n_chunks * _CHUNK == per_w

    mesh = plsc.VectorSubcoreMesh(
        core_axis_name="c", subcore_axis_name="s",
        num_cores=_NC, num_subcores=_NS,
    )

    @functools.partial(
        pl.kernel,
        out_type=(
            jax.ShapeDtypeStruct((_NW, _L), jnp.int32),
            jax.ShapeDtypeStruct((_NW, _L), jnp.int32),
        ),
        mesh=mesh,
        scratch_types=[
            pltpu.VMEM((_CHUNK,), jnp.int32),
            pltpu.VMEM((_CHUNK,), jnp.int32),
            pltpu.VMEM((_CHUNK,), jnp.int32),
            pltpu.VMEM((_CHUNK,), jnp.int32),
            pltpu.VMEM((_L,), jnp.int32),
            pltpu.VMEM((_L,), jnp.int32),
            pltpu.SemaphoreType.DMA,
            pltpu.SemaphoreType.DMA,
            pltpu.SemaphoreType.DMA,
            pltpu.SemaphoreType.DMA,
        ],
    )
    def sc_partials(t_hbm, p_hbm, out_t, out_or,
                    tb0, tb1, pb0, pb1, acc_t_v, acc_or_v,
                    st0, st1, sp0, sp1):
        cid = lax.axis_index("c")
        sid = lax.axis_index("s")
        wid = sid * _NC + cid
        base = wid * per_w

        tbufs = (tb0, tb1)
        pbufs = (pb0, pb1)
        tsems = (st0, st1)
        psems = (sp0, sp1)

        def start(chunk, slot):
            off = base + chunk * _CHUNK
            dt = pltpu.async_copy(
                t_hbm.at[pl.ds(off, _CHUNK)], tbufs[slot], tsems[slot])
            dp = pltpu.async_copy(
                p_hbm.at[pl.ds(off, _CHUNK)], pbufs[slot], psems[slot])
            return dt, dp

        pending = [None, None]
        pending[0] = start(0, 0)

        z = jnp.zeros((_L,), jnp.int32)
        acc = (z, z, z, z)
        for c in range(n_chunks):
            slot = c & 1
            nxt = slot ^ 1
            if c + 1 < n_chunks:
                pending[nxt] = start(c + 1, nxt)
            dt, dp = pending[slot]
            dt.wait()
            dp.wait()

            tb = tbufs[slot]
            pb = pbufs[slot]

            def body(i, carry):
                at0, ao0, at1, ao1 = carry
                t0 = tb[pl.ds(i, _L)]
                p0 = pb[pl.ds(i, _L)]
                t1 = tb[pl.ds(i + _L, _L)]
                p1 = pb[pl.ds(i + _L, _L)]
                return (at0 + t0, ao0 + (t0 | p0),
                        at1 + t1, ao1 + (t1 | p1))

            acc = plsc.parallel_loop(
                0, _CHUNK, 2 * _L, unroll=8, carry=acc)(body)

        acc_t_v[...] = acc[0] + acc[2]
        acc_or_v[...] = acc[1] + acc[3]
        pltpu.sync_copy(acc_t_v, out_t.at[wid])
        pltpu.sync_copy(acc_or_v, out_or.at[wid])

    return sc_partials


def _tc_partials(t, p, e0):
    n_el = t.shape[0] - e0
    n_blocks = n_el // _TC_BLOCK
    assert n_blocks * _TC_BLOCK == n_el and e0 % _TC_BLOCK == 0

    def body(t_ref, p_ref, out_ref, acc_ref):
        i = pl.program_id(0)
        t_blk = t_ref[...]
        p_blk = p_ref[...]
        st = jnp.sum(t_blk)
        so = jnp.sum(t_blk | p_blk)

        @pl.when(i == 0)
        def _init():
            acc_ref[0] = st
            acc_ref[1] = so

        @pl.when(i > 0)
        def _acc():
            acc_ref[0] += st
            acc_ref[1] += so

        @pl.when(i == n_blocks - 1)
        def _out():
            out_ref[0] = acc_ref[0]
            out_ref[1] = acc_ref[1]

    blk = lambda i: (e0 // _TC_BLOCK + i,)
    return pl.pallas_call(
        body,
        grid=(n_blocks,),
        in_specs=[
            pl.BlockSpec((_TC_BLOCK,), blk),
            pl.BlockSpec((_TC_BLOCK,), blk),
        ],
        out_specs=pl.BlockSpec(memory_space=pltpu.SMEM),
        out_shape=jax.ShapeDtypeStruct((2,), jnp.int32),
        scratch_shapes=[pltpu.SMEM((2,), jnp.int32)],
    )(t, p)


def _finish(sc_pt, sc_po, tc_sums, n):
    def body(pt_ref, po_ref, tc_ref, out_ref):
        st = jnp.sum(pt_ref[...]) + tc_ref[0]
        so = jnp.sum(po_ref[...]) + tc_ref[1]
        tn = (n - so).astype(jnp.float32)
        tnfp = (n - st).astype(jnp.float32)
        out_ref[...] = jnp.full((1, 1), tn / tnfp, jnp.float32)

    out = pl.pallas_call(
        body,
        in_specs=[
            pl.BlockSpec(memory_space=pltpu.VMEM),
            pl.BlockSpec(memory_space=pltpu.VMEM),
            pl.BlockSpec(memory_space=pltpu.SMEM),
        ],
        out_shape=jax.ShapeDtypeStruct((1, 1), jnp.float32),
    )(sc_pt, sc_po, tc_sums)
    return out[0, 0]


@jax.jit
def kernel(y_true, y_pred):
    n = y_true.shape[0]
    t = y_true.astype(jnp.int32)
    p = y_pred.astype(jnp.int32)
    sc_pt, sc_po = _make_sc_partials(_F)(t, p)
    tc_sums = _tc_partials(t, p, _F)
    return _finish(sc_pt, sc_po, tc_sums, n)


# speedup vs baseline: 1.2398x; 1.0569x over previous
import functools

import jax
import jax.numpy as jnp
from jax import lax
from jax.experimental import pallas as pl
from jax.experimental.pallas import tpu as pltpu
from jax.experimental.pallas import tpu_sc as plsc

_NC = 2
_NS = 16
_L = 16
_NW = _NC * _NS
_CHUNK = 16384

_N = 16777216
_F = 9437184
_TC_BLOCK = 524288


def _make_sc_partials(n_sc):
    per_w = n_sc // _NW
    n_chunks = per_w // _CHUNK
    assert per_w * _NW == n_sc and n_chunks * _CHUNK == per_w

    mesh = plsc.VectorSubcoreMesh(
        core_axis_name="c", subcore_axis_name="s",
        num_cores=_NC, num_subcores=_NS,
    )

    @functools.partial(
        pl.kernel,
        out_type=(
            jax.ShapeDtypeStruct((_NW, _L), jnp.int32),
            jax.ShapeDtypeStruct((_NW, _L), jnp.int32),
        ),
        mesh=mesh,
        scratch_types=[
            pltpu.VMEM((_CHUNK,), jnp.int32),
            pltpu.VMEM((_CHUNK,), jnp.int32),
            pltpu.VMEM((_CHUNK,), jnp.int32),
            pltpu.VMEM((_CHUNK,), jnp.int32),
            pltpu.VMEM((_L,), jnp.int32),
            pltpu.VMEM((_L,), jnp.int32),
            pltpu.SemaphoreType.DMA,
            pltpu.SemaphoreType.DMA,
            pltpu.SemaphoreType.DMA,
            pltpu.SemaphoreType.DMA,
        ],
    )
    def sc_partials(t_hbm, p_hbm, out_t, out_or,
                    tb0, tb1, pb0, pb1, acc_t_v, acc_or_v,
                    st0, st1, sp0, sp1):
        cid = lax.axis_index("c")
        sid = lax.axis_index("s")
        wid = sid * _NC + cid
        base = wid * per_w

        tbufs = (tb0, tb1)
        pbufs = (pb0, pb1)
        tsems = (st0, st1)
        psems = (sp0, sp1)

        def start(chunk, slot):
            off = base + chunk * _CHUNK
            dt = pltpu.async_copy(
                t_hbm.at[pl.ds(off, _CHUNK)], tbufs[slot], tsems[slot])
            dp = pltpu.async_copy(
                p_hbm.at[pl.ds(off, _CHUNK)], pbufs[slot], psems[slot])
            return dt, dp

        pending = [None, None]
        pending[0] = start(0, 0)

        z = jnp.zeros((_L,), jnp.int32)
        acc = (z, z, z, z)
        for c in range(n_chunks):
            slot = c & 1
            nxt = slot ^ 1
            if c + 1 < n_chunks:
                pending[nxt] = start(c + 1, nxt)
            dt, dp = pending[slot]
            dt.wait()
            dp.wait()

            tb = tbufs[slot]
            pb = pbufs[slot]

            def body(i, carry):
                at0, ao0, at1, ao1 = carry
                t0 = tb[pl.ds(i, _L)]
                p0 = pb[pl.ds(i, _L)]
                t1 = tb[pl.ds(i + _L, _L)]
                p1 = pb[pl.ds(i + _L, _L)]
                return (at0 + t0, ao0 + (t0 | p0),
                        at1 + t1, ao1 + (t1 | p1))

            acc = plsc.parallel_loop(
                0, _CHUNK, 2 * _L, unroll=8, carry=acc)(body)

        acc_t_v[...] = acc[0] + acc[2]
        acc_or_v[...] = acc[1] + acc[3]
        pltpu.sync_copy(acc_t_v, out_t.at[wid])
        pltpu.sync_copy(acc_or_v, out_or.at[wid])

    return sc_partials


def _tc_partials(t, p, e0):
    n_el = t.shape[0] - e0
    n_blocks = n_el // _TC_BLOCK
    assert n_blocks * _TC_BLOCK == n_el and e0 % _TC_BLOCK == 0

    def body(t_ref, p_ref, out_ref, acc_ref):
        i = pl.program_id(0)
        t_blk = t_ref[...]
        p_blk = p_ref[...]
        st = jnp.sum(t_blk)
        so = jnp.sum(t_blk | p_blk)

        @pl.when(i == 0)
        def _init():
            acc_ref[0] = st
            acc_ref[1] = so

        @pl.when(i > 0)
        def _acc():
            acc_ref[0] += st
            acc_ref[1] += so

        @pl.when(i == n_blocks - 1)
        def _out():
            out_ref[0] = acc_ref[0]
            out_ref[1] = acc_ref[1]

    blk = lambda i: (e0 // _TC_BLOCK + i,)
    return pl.pallas_call(
        body,
        grid=(n_blocks,),
        in_specs=[
            pl.BlockSpec((_TC_BLOCK,), blk),
            pl.BlockSpec((_TC_BLOCK,), blk),
        ],
        out_specs=pl.BlockSpec(memory_space=pltpu.SMEM),
        out_shape=jax.ShapeDtypeStruct((2,), jnp.int32),
        scratch_shapes=[pltpu.SMEM((2,), jnp.int32)],
    )(t, p)


def _finish(sc_pt, sc_po, tc_sums, n):
    def body(pt_ref, po_ref, tc_ref, out_ref):
        st = jnp.sum(pt_ref[...]) + tc_ref[0]
        so = jnp.sum(po_ref[...]) + tc_ref[1]
        tn = (n - so).astype(jnp.float32)
        tnfp = (n - st).astype(jnp.float32)
        out_ref[...] = jnp.full((1, 1), tn / tnfp, jnp.float32)

    out = pl.pallas_call(
        body,
        in_specs=[
            pl.BlockSpec(memory_space=pltpu.VMEM),
            pl.BlockSpec(memory_space=pltpu.VMEM),
            pl.BlockSpec(memory_space=pltpu.SMEM),
        ],
        out_shape=jax.ShapeDtypeStruct((1, 1), jnp.float32),
    )(sc_pt, sc_po, tc_sums)
    return out[0, 0]


@jax.jit
def kernel(y_true, y_pred):
    n = y_true.shape[0]
    t = y_true.astype(jnp.int32)
    p = y_pred.astype(jnp.int32)
    sc_pt, sc_po = _make_sc_partials(_F)(t, p)
    tc_sums = _tc_partials(t, p, _F)
    return _finish(sc_pt, sc_po, tc_sums, n)
